# baseline (speedup 1.0000x reference)
import functools

import jax
import jax.numpy as jnp
from jax import lax
from jax.experimental import pallas as pl
from jax.experimental.pallas import tpu as pltpu
from jax.experimental.pallas import tpu_sc as plsc

_info = plsc.get_sparse_core_info()
_NC = _info.num_cores
_NS = _info.num_subcores


def _make_sc_broadcast(B, S, D, dtype):
    rows_per_core = S // _NC
    N_TILE_W = _NS - 1
    TILE_ROWS = 160
    SPM_ROWS = rows_per_core - N_TILE_W * TILE_ROWS
    CH = 32
    NBUF = 3
    n_steps = TILE_ROWS // CH
    SCH = 32
    SNBUF = 4
    sn_steps = SPM_ROWS // SCH
    mesh = plsc.VectorSubcoreMesh(core_axis_name="c", subcore_axis_name="s")

    @functools.partial(
        pl.kernel,
        mesh=mesh,
        out_type=jax.ShapeDtypeStruct((B, S, D), dtype),
        scratch_types=[pltpu.VMEM((NBUF, CH, D), dtype)]
        + [pltpu.SemaphoreType.DMA] * (2 * NBUF)
        + [pltpu.VMEM_SHARED((SNBUF, SCH, D), dtype)]
        + [pltpu.SemaphoreType.DMA] * (2 * SNBUF),
    )
    def sc_kernel(table_hbm, out_hbm, buf, *rest):
        rsem = rest[:NBUF]
        wsem = rest[NBUF : 2 * NBUF]
        sbuf = rest[2 * NBUF]
        srsem = rest[2 * NBUF + 1 : 2 * NBUF + 1 + SNBUF]
        swsem = rest[2 * NBUF + 1 + SNBUF :]
        c = lax.axis_index("c")
        t = lax.axis_index("s")
        core_base = c * rows_per_core

        @pl.when(t > 0)
        def _tile_path():
            base = core_base + (t - 1) * TILE_ROWS
            reads = {}
            writes = {}
            reads[0] = pltpu.async_copy(
                table_hbm.at[pl.ds(base, CH)], buf.at[0], rsem[0]
            )
            for i in range(n_steps):
                reads.pop(i).wait()
                nxt = i + 1
                if nxt < n_steps:
                    prev = nxt - NBUF
                    if prev in writes:
                        for h in writes.pop(prev):
                            h.wait()
                    reads[nxt] = pltpu.async_copy(
                        table_hbm.at[pl.ds(base + nxt * CH, CH)],
                        buf.at[nxt % NBUF],
                        rsem[nxt % NBUF],
                    )
                r0 = base + i * CH
                writes[i] = [
                    pltpu.async_copy(
                        buf.at[i % NBUF],
                        out_hbm.at[b, pl.ds(r0, CH)],
                        wsem[i % NBUF],
                    )
                    for b in range(B)
                ]
            for hs in writes.values():
                for h in hs:
                    h.wait()

        @pl.when(t == 0)
        def _spmem_path():
            sbase = core_base + N_TILE_W * TILE_ROWS
            sreads = {}
            swrites = {}
            sreads[0] = pltpu.async_copy(
                table_hbm.at[pl.ds(sbase, SCH)], sbuf.at[0], srsem[0]
            )
            for i in range(sn_steps):
                sreads.pop(i).wait()
                nxt = i + 1
                if nxt < sn_steps:
                    prev = nxt - SNBUF
                    if prev in swrites:
                        for h in swrites.pop(prev):
                            h.wait()
                    sreads[nxt] = pltpu.async_copy(
                        table_hbm.at[pl.ds(sbase + nxt * SCH, SCH)],
                        sbuf.at[nxt % SNBUF],
                        srsem[nxt % SNBUF],
                    )
                r0 = sbase + i * SCH
                swrites[i] = [
                    pltpu.async_copy(
                        sbuf.at[i % SNBUF],
                        out_hbm.at[b, pl.ds(r0, SCH)],
                        swsem[i % SNBUF],
                    )
                    for b in range(B)
                ]
            for hs in swrites.values():
                for h in hs:
                    h.wait()

    return sc_kernel


def kernel(inputs, table):
    B = inputs.shape[0]
    S, D = table.shape
    return _make_sc_broadcast(B, S, D, table.dtype)(table)


# speedup vs baseline: 1.2727x; 1.2727x over previous
import functools

import jax
import jax.numpy as jnp
from jax import lax
from jax.experimental import pallas as pl
from jax.experimental.pallas import tpu as pltpu
from jax.experimental.pallas import tpu_sc as plsc

_info = plsc.get_sparse_core_info()
_NC = _info.num_cores
_NS = _info.num_subcores


def _make_sc_broadcast(B, S, D, dtype):
    rows_per_core = S // _NC
    N_TILE_W = _NS - 1
    TILE_ROWS = 128
    SPM_ROWS = rows_per_core - N_TILE_W * TILE_ROWS
    CH = 32
    NBUF = 3
    n_steps = TILE_ROWS // CH
    SCH = 128
    SNBUF = 4
    sn_steps = SPM_ROWS // SCH
    mesh = plsc.VectorSubcoreMesh(core_axis_name="c", subcore_axis_name="s")

    @functools.partial(
        pl.kernel,
        mesh=mesh,
        out_type=jax.ShapeDtypeStruct((B, S, D), dtype),
        scratch_types=[pltpu.VMEM((NBUF, CH, D), dtype)]
        + [pltpu.SemaphoreType.DMA] * (2 * NBUF)
        + [pltpu.VMEM_SHARED((SNBUF, SCH, D), dtype)]
        + [pltpu.SemaphoreType.DMA] * (2 * SNBUF),
    )
    def sc_kernel(table_hbm, out_hbm, buf, *rest):
        rsem = rest[:NBUF]
        wsem = rest[NBUF : 2 * NBUF]
        sbuf = rest[2 * NBUF]
        srsem = rest[2 * NBUF + 1 : 2 * NBUF + 1 + SNBUF]
        swsem = rest[2 * NBUF + 1 + SNBUF :]
        c = lax.axis_index("c")
        t = lax.axis_index("s")
        core_base = c * rows_per_core

        @pl.when(t > 0)
        def _tile_path():
            base = core_base + (t - 1) * TILE_ROWS
            reads = {}
            writes = {}
            reads[0] = pltpu.async_copy(
                table_hbm.at[pl.ds(base, CH)], buf.at[0], rsem[0]
            )
            for i in range(n_steps):
                reads.pop(i).wait()
                nxt = i + 1
                if nxt < n_steps:
                    prev = nxt - NBUF
                    if prev in writes:
                        for h in writes.pop(prev):
                            h.wait()
                    reads[nxt] = pltpu.async_copy(
                        table_hbm.at[pl.ds(base + nxt * CH, CH)],
                        buf.at[nxt % NBUF],
                        rsem[nxt % NBUF],
                    )
                r0 = base + i * CH
                writes[i] = [
                    pltpu.async_copy(
                        buf.at[i % NBUF],
                        out_hbm.at[b, pl.ds(r0, CH)],
                        wsem[i % NBUF],
                    )
                    for b in range(B)
                ]
            for hs in writes.values():
                for h in hs:
                    h.wait()

        @pl.when(t == 0)
        def _spmem_path():
            sbase = core_base + N_TILE_W * TILE_ROWS
            sreads = {}
            swrites = {}
            sreads[0] = pltpu.async_copy(
                table_hbm.at[pl.ds(sbase, SCH)], sbuf.at[0], srsem[0]
            )
            for i in range(sn_steps):
                sreads.pop(i).wait()
                nxt = i + 1
                if nxt < sn_steps:
                    prev = nxt - SNBUF
                    if prev in swrites:
                        for h in swrites.pop(prev):
                            h.wait()
                    sreads[nxt] = pltpu.async_copy(
                        table_hbm.at[pl.ds(sbase + nxt * SCH, SCH)],
                        sbuf.at[nxt % SNBUF],
                        srsem[nxt % SNBUF],
                    )
                r0 = sbase + i * SCH
                swrites[i] = [
                    pltpu.async_copy(
                        sbuf.at[i % SNBUF],
                        out_hbm.at[b, pl.ds(r0, SCH)],
                        swsem[i % SNBUF],
                    )
                    for b in range(B)
                ]
            for hs in swrites.values():
                for h in hs:
                    h.wait()

    return sc_kernel


def kernel(inputs, table):
    B = inputs.shape[0]
    S, D = table.shape
    return _make_sc_broadcast(B, S, D, table.dtype)(table)


# speedup vs baseline: 1.2804x; 1.0061x over previous
import functools

import jax
import jax.numpy as jnp
from jax import lax
from jax.experimental import pallas as pl
from jax.experimental.pallas import tpu as pltpu
from jax.experimental.pallas import tpu_sc as plsc

_info = plsc.get_sparse_core_info()
_NC = _info.num_cores
_NS = _info.num_subcores


def _make_sc_broadcast(B, S, D, dtype):
    rows_per_core = S // _NC
    N_TILE_W = _NS - 1
    TILE_ROWS = 176
    SPM_ROWS = rows_per_core - N_TILE_W * TILE_ROWS
    CH = 16
    NBUF = 4
    n_steps = TILE_ROWS // CH
    SCH = 112
    SNBUF = 4
    sn_steps = SPM_ROWS // SCH
    mesh = plsc.VectorSubcoreMesh(core_axis_name="c", subcore_axis_name="s")

    @functools.partial(
        pl.kernel,
        mesh=mesh,
        out_type=jax.ShapeDtypeStruct((B, S, D), dtype),
        scratch_types=[pltpu.VMEM((NBUF, CH, D), dtype)]
        + [pltpu.SemaphoreType.DMA] * (2 * NBUF)
        + [pltpu.VMEM_SHARED((SNBUF, SCH, D), dtype)]
        + [pltpu.SemaphoreType.DMA] * (2 * SNBUF),
    )
    def sc_kernel(table_hbm, out_hbm, buf, *rest):
        rsem = rest[:NBUF]
        wsem = rest[NBUF : 2 * NBUF]
        sbuf = rest[2 * NBUF]
        srsem = rest[2 * NBUF + 1 : 2 * NBUF + 1 + SNBUF]
        swsem = rest[2 * NBUF + 1 + SNBUF :]
        c = lax.axis_index("c")
        t = lax.axis_index("s")
        core_base = c * rows_per_core

        @pl.when(t > 0)
        def _tile_path():
            base = core_base + (t - 1) * TILE_ROWS
            reads = {}
            writes = {}
            reads[0] = pltpu.async_copy(
                table_hbm.at[pl.ds(base, CH)], buf.at[0], rsem[0]
            )
            for i in range(n_steps):
                reads.pop(i).wait()
                nxt = i + 1
                if nxt < n_steps:
                    prev = nxt - NBUF
                    if prev in writes:
                        for h in writes.pop(prev):
                            h.wait()
                    reads[nxt] = pltpu.async_copy(
                        table_hbm.at[pl.ds(base + nxt * CH, CH)],
                        buf.at[nxt % NBUF],
                        rsem[nxt % NBUF],
                    )
                r0 = base + i * CH
                writes[i] = [
                    pltpu.async_copy(
                        buf.at[i % NBUF],
                        out_hbm.at[b, pl.ds(r0, CH)],
                        wsem[i % NBUF],
                    )
                    for b in range(B)
                ]
            for hs in writes.values():
                for h in hs:
                    h.wait()

        @pl.when(t == 0)
        def _spmem_path():
            sbase = core_base + N_TILE_W * TILE_ROWS
            sreads = {}
            swrites = {}
            sreads[0] = pltpu.async_copy(
                table_hbm.at[pl.ds(sbase, SCH)], sbuf.at[0], srsem[0]
            )
            for i in range(sn_steps):
                sreads.pop(i).wait()
                nxt = i + 1
                if nxt < sn_steps:
                    prev = nxt - SNBUF
                    if prev in swrites:
                        for h in swrites.pop(prev):
                            h.wait()
                    sreads[nxt] = pltpu.async_copy(
                        table_hbm.at[pl.ds(sbase + nxt * SCH, SCH)],
                        sbuf.at[nxt % SNBUF],
                        srsem[nxt % SNBUF],
                    )
                r0 = sbase + i * SCH
                swrites[i] = [
                    pltpu.async_copy(
                        sbuf.at[i % SNBUF],
                        out_hbm.at[b, pl.ds(r0, SCH)],
                        swsem[i % SNBUF],
                    )
                    for b in range(B)
                ]
            for hs in swrites.values():
                for h in hs:
                    h.wait()

    return sc_kernel


def kernel(inputs, table):
    B = inputs.shape[0]
    S, D = table.shape
    return _make_sc_broadcast(B, S, D, table.dtype)(table)
